# baseline (speedup 1.0000x reference)
import functools

import jax
import jax.numpy as jnp
from jax import lax
from jax.experimental import pallas as pl
from jax.experimental.pallas import tpu as pltpu
from jax.experimental.pallas import tpu_sc as plsc

_T = 16384
_DIM = 2048
_NE = 8
_TOPK = 2
_N_GROUPS = 2
_GSIZE = _NE // _N_GROUPS
_ROUTE_SCALE = 2.5
_BT = 1024
_NC = 2
_NS = 16
_NW = _NC * _NS
_TPW = _T // _NW
_L = 16


def _scores_block(x_ref, w_ref, st_ref):
    logits = jax.lax.dot_general(
        x_ref[...], w_ref[...], (((1,), (1,)), ((), ())),
        preferred_element_type=jnp.float32,
    )
    st_ref[...] = 1.0 / (1.0 + jnp.exp(-logits.T))


def _tc_scores(x, weight):
    return pl.pallas_call(
        _scores_block,
        grid=(_T // _BT,),
        in_specs=[
            pl.BlockSpec((_BT, _DIM), lambda i: (i, 0)),
            pl.BlockSpec((_NE, _DIM), lambda i: (0, 0)),
        ],
        out_specs=pl.BlockSpec((_NE, _BT), lambda i: (0, i)),
        out_shape=jax.ShapeDtypeStruct((_NE, _T), jnp.float32),
        compiler_params=pltpu.CompilerParams(
            dimension_semantics=("arbitrary",),
        ),
    )(x, weight)


@functools.partial(
    pl.kernel,
    mesh=plsc.VectorSubcoreMesh(core_axis_name="c", subcore_axis_name="s"),
    out_type=[
        jax.ShapeDtypeStruct((_TOPK, _T), jnp.float32),
        jax.ShapeDtypeStruct((_TOPK, _T), jnp.int32),
    ],
    scratch_types=[
        pltpu.VMEM((_NE, _TPW), jnp.float32),
        pltpu.VMEM((_TOPK, _TPW), jnp.float32),
        pltpu.VMEM((_TOPK, _TPW), jnp.int32),
        pltpu.SemaphoreType.DMA,
        pltpu.SemaphoreType.DMA,
    ],
)
def _sc_route(st_hbm, wout_hbm, iout_hbm, sv, wv, iv, sem_w, sem_i):
    wid = lax.axis_index("s") * _NC + lax.axis_index("c")
    base = wid * _TPW
    pltpu.sync_copy(st_hbm.at[:, pl.ds(base, _TPW)], sv)
    neg = jnp.full((_L,), -jnp.inf, jnp.float32)

    def chunk(c):
        o = c * _L
        s = [sv[e, pl.ds(o, _L)] for e in range(_NE)]
        g0 = jnp.maximum(jnp.maximum(s[0], s[1]), jnp.maximum(s[2], s[3]))
        g1 = jnp.maximum(jnp.maximum(s[4], s[5]), jnp.maximum(s[6], s[7]))
        use0 = g0 >= g1
        t = [jnp.where(use0, s[e], neg) if e < _GSIZE
             else jnp.where(use0, neg, s[e]) for e in range(_NE)]
        v1 = t[0]
        i1 = jnp.zeros((_L,), jnp.int32)
        for e in range(1, _NE):
            gt = t[e] > v1
            v1 = jnp.where(gt, t[e], v1)
            i1 = jnp.where(gt, jnp.int32(e), i1)
        t = [jnp.where(i1 == e, neg, t[e]) for e in range(_NE)]
        v2 = t[0]
        i2 = jnp.zeros((_L,), jnp.int32)
        for e in range(1, _NE):
            gt = t[e] > v2
            v2 = jnp.where(gt, t[e], v2)
            i2 = jnp.where(gt, jnp.int32(e), i2)
        scale = _ROUTE_SCALE / (v1 + v2)
        wv[0, pl.ds(o, _L)] = v1 * scale
        wv[1, pl.ds(o, _L)] = v2 * scale
        iv[0, pl.ds(o, _L)] = i1
        iv[1, pl.ds(o, _L)] = i2

    for c in range(_TPW // _L):
        chunk(c)
    cw = pltpu.async_copy(wv, wout_hbm.at[:, pl.ds(base, _TPW)], sem_w)
    ci = pltpu.async_copy(iv, iout_hbm.at[:, pl.ds(base, _TPW)], sem_i)
    cw.wait()
    ci.wait()


@jax.jit
def kernel(x, weight):
    st = _tc_scores(x, weight)
    wout, iout = _sc_route(st)
    return wout.T.astype(x.dtype), iout.T


# speedup vs baseline: 1.0048x; 1.0048x over previous
import functools

import jax
import jax.numpy as jnp
from jax import lax
from jax.experimental import pallas as pl
from jax.experimental.pallas import tpu as pltpu
from jax.experimental.pallas import tpu_sc as plsc

_T = 16384
_DIM = 2048
_NE = 8
_TOPK = 2
_N_GROUPS = 2
_GSIZE = _NE // _N_GROUPS
_ROUTE_SCALE = 2.5
_BT = 1024
_NC = 2
_NS = 16
_NW = _NC * _NS
_TPW = _T // _NW
_L = 16


def _scores_block(x_ref, w_ref, st_ref):
    logits = jax.lax.dot_general(
        x_ref[...], w_ref[...], (((1,), (1,)), ((), ())),
        preferred_element_type=jnp.float32,
    )
    st_ref[...] = 1.0 / (1.0 + jnp.exp(-logits.T))


def _tc_scores(x, weight):
    return pl.pallas_call(
        _scores_block,
        grid=(_T // _BT,),
        in_specs=[
            pl.BlockSpec((_BT, _DIM), lambda i: (i, 0)),
            pl.BlockSpec((_NE, _DIM), lambda i: (0, 0)),
        ],
        out_specs=pl.BlockSpec((_NE, _BT), lambda i: (0, i)),
        out_shape=jax.ShapeDtypeStruct((_NE, _T), jnp.float32),
        compiler_params=pltpu.CompilerParams(
            dimension_semantics=("arbitrary",),
        ),
    )(x, weight)


@functools.partial(
    pl.kernel,
    mesh=plsc.VectorSubcoreMesh(core_axis_name="c", subcore_axis_name="s"),
    out_type=[
        jax.ShapeDtypeStruct((_TOPK, _T), jnp.float32),
        jax.ShapeDtypeStruct((_TOPK, _T), jnp.int32),
    ],
    scratch_types=[
        pltpu.VMEM((_NE, _TPW), jnp.float32),
        pltpu.VMEM((_TOPK, _TPW), jnp.float32),
        pltpu.VMEM((_TOPK, _TPW), jnp.int32),
    ],
)
def _sc_route(st_hbm, wout_hbm, iout_hbm, sv, wv, iv):
    wid = lax.axis_index("s") * _NC + lax.axis_index("c")
    base = wid * _TPW
    pltpu.sync_copy(st_hbm.at[:, pl.ds(base, _TPW)], sv)
    neg = jnp.full((_L,), -jnp.inf, jnp.float32)

    def chunk(c):
        o = c * _L
        s = [sv[e, pl.ds(o, _L)] for e in range(_NE)]
        g0 = jnp.maximum(jnp.maximum(s[0], s[1]), jnp.maximum(s[2], s[3]))
        g1 = jnp.maximum(jnp.maximum(s[4], s[5]), jnp.maximum(s[6], s[7]))
        use0 = g0 >= g1
        t = [jnp.where(use0, s[e], neg) if e < _GSIZE
             else jnp.where(use0, neg, s[e]) for e in range(_NE)]
        v1 = t[0]
        i1 = jnp.zeros((_L,), jnp.int32)
        for e in range(1, _NE):
            gt = t[e] > v1
            v1 = jnp.where(gt, t[e], v1)
            i1 = jnp.where(gt, jnp.int32(e), i1)
        t = [jnp.where(i1 == e, neg, t[e]) for e in range(_NE)]
        v2 = t[0]
        i2 = jnp.zeros((_L,), jnp.int32)
        for e in range(1, _NE):
            gt = t[e] > v2
            v2 = jnp.where(gt, t[e], v2)
            i2 = jnp.where(gt, jnp.int32(e), i2)
        scale = _ROUTE_SCALE / (v1 + v2)
        wv[0, pl.ds(o, _L)] = v1 * scale
        wv[1, pl.ds(o, _L)] = v2 * scale
        iv[0, pl.ds(o, _L)] = i1
        iv[1, pl.ds(o, _L)] = i2

    for c in range(_TPW // _L):
        chunk(c)
    pltpu.sync_copy(wv, wout_hbm.at[:, pl.ds(base, _TPW)])
    pltpu.sync_copy(iv, iout_hbm.at[:, pl.ds(base, _TPW)])


@jax.jit
def kernel(x, weight):
    st = _tc_scores(x, weight)
    wout, iout = _sc_route(st)
    return wout.T.astype(x.dtype), iout.T
